# baseline (speedup 1.0000x reference)
import functools

import jax
import jax.numpy as jnp
from jax import lax
from jax.experimental import pallas as pl
from jax.experimental.pallas import tpu as pltpu
from jax.experimental.pallas import tpu_sc as plsc

E = 160000
N = 10000
D = 256
BE = 3200
NBLK = E // BE

NC = 2
NS = 16
NW = NC * NS
NPAD = 10240
NT = NPAD // NW
ACCW = (NT + 1) * D
K = 64
EK = E - K


def _mlp_kernel(xi_ref, xj_ref, ea_ref, w_ref, b_ref, m_ref):
    xi = xi_ref[...].astype(jnp.bfloat16)
    xj = xj_ref[...].astype(jnp.bfloat16)
    ea = ea_ref[...].astype(jnp.bfloat16)
    w = w_ref[...].astype(jnp.bfloat16)
    acc = jax.lax.dot_general(xi, w[0:D, :], (((1,), (0,)), ((), ())),
                              preferred_element_type=jnp.float32)
    acc += jax.lax.dot_general(xj, w[D:2 * D, :], (((1,), (0,)), ((), ())),
                               preferred_element_type=jnp.float32)
    acc += jax.lax.dot_general(ea, w[2 * D:3 * D, :], (((1,), (0,)), ((), ())),
                               preferred_element_type=jnp.float32)
    m_ref[...] = jnp.maximum(acc + b_ref[...], 0.0)


def _segsum_body(m_hbm, rec_hbm, bounds_hbm, aggr_hbm,
                 accf, stage, idsv, bv, dma_sem):
    core = lax.axis_index("c")
    sub = lax.axis_index("s")
    wid = sub * NC + core
    base = wid * NT

    pltpu.sync_copy(bounds_hbm, bv)
    vb = bv[pl.ds(2 * wid, 16)]
    sw = vb[0]
    ew = vb[1]
    s_al = (sw // 8) * 8
    nch = lax.div(ew - s_al + (K - 1), K)

    zv = jnp.zeros((16,), jnp.float32)
    iota16 = lax.iota(jnp.int32, 16)

    def zero_body(r, carry):
        for j in range(16):
            accf[pl.ds(r * D + j * 16, 16)] = zv
        return carry

    lax.fori_loop(0, NT + 1, zero_body, 0)

    def chunk_body(c, carry):
        start_u = s_al + c * K
        sc = jnp.minimum(start_u, EK)
        pltpu.sync_copy(m_hbm.at[pl.ds(sc, K)], stage)
        pltpu.sync_copy(rec_hbm.at[pl.ds(sc, K)], idsv)
        lo = jnp.maximum(sw, start_u)
        for g in range(K // 16):
            idg = idsv[pl.ds(g * 16, 16)]
            gidx = sc + g * 16 + iota16
            valid = (gidx >= lo) & (gidx < ew)
            rel = jnp.where(valid, idg - base, NT)
            rel_off = rel * D
            for e in range(16):
                t = rel_off[e]
                for k in range(16):
                    x = stage[g * 16 + e, pl.ds(k * 16, 16)]
                    plsc.addupdate(accf.at[pl.ds(t + k * 16, 16)], x)
        return carry

    lax.fori_loop(0, nch, chunk_body, 0)
    pltpu.sync_copy(accf.at[pl.ds(0, NT * D)],
                    aggr_hbm.at[pl.ds(base * D, NT * D)])


@jax.jit
def _run(x_i, x_j, recipients, edge_attr, W, b):
    m = pl.pallas_call(
        _mlp_kernel,
        grid=(NBLK,),
        in_specs=[
            pl.BlockSpec((BE, D), lambda i: (i, 0)),
            pl.BlockSpec((BE, D), lambda i: (i, 0)),
            pl.BlockSpec((BE, D), lambda i: (i, 0)),
            pl.BlockSpec((3 * D, D), lambda i: (0, 0)),
            pl.BlockSpec((1, D), lambda i: (0, 0)),
        ],
        out_specs=pl.BlockSpec((BE, D), lambda i: (i, 0)),
        out_shape=jax.ShapeDtypeStruct((E, D), jnp.float32),
    )(x_i, x_j, edge_attr, W, b.reshape(1, D))

    node_edges = jnp.searchsorted(
        recipients, jnp.arange(0, NPAD + NT, NT, dtype=jnp.int32)
    ).astype(jnp.int32)
    bounds = jnp.stack(
        [node_edges[:-1], node_edges[1:]], axis=1).reshape(2 * NW)
    bounds = jnp.pad(bounds, (0, 16))

    mesh = plsc.VectorSubcoreMesh(core_axis_name="c", subcore_axis_name="s")
    segsum = pl.kernel(
        _segsum_body,
        out_type=jax.ShapeDtypeStruct((NPAD * D,), jnp.float32),
        mesh=mesh,
        scratch_types=[
            pltpu.VMEM((ACCW,), jnp.float32),
            pltpu.VMEM((K, D), jnp.float32),
            pltpu.VMEM((K,), jnp.int32),
            pltpu.VMEM((2 * NW + 16,), jnp.int32),
            pltpu.SemaphoreType.DMA,
        ],
    )
    aggr = segsum(m, recipients, bounds).reshape(NPAD, D)
    return aggr[:N], m


def kernel(x_i, x_j, recipients, edge_attr, num_segments, W, b):
    aggr, m = _run(x_i, x_j, recipients, edge_attr, W, b)
    return (aggr, m)


# speedup vs baseline: 3.8508x; 3.8508x over previous
import functools

import jax
import jax.numpy as jnp
from jax.experimental import pallas as pl
from jax.experimental.pallas import tpu as pltpu

E = 160000
N = 10000
D = 256
BE = 4000
C = 128
NBLK = E // BE
NPAD = ((N + C - 1) // C) * C


def _fused_kernel(cstart_ref, nch_ref, xi_ref, xj_ref, ea_ref, w_ref, b_ref,
                  rec_ref, m_ref, aggr_ref):
    i = pl.program_id(0)

    @pl.when(i == 0)
    def _init():
        aggr_ref[...] = jnp.zeros_like(aggr_ref)

    xi = xi_ref[...].astype(jnp.bfloat16)
    xj = xj_ref[...].astype(jnp.bfloat16)
    ea = ea_ref[...].astype(jnp.bfloat16)
    w = w_ref[...].astype(jnp.bfloat16)
    acc = jax.lax.dot_general(xi, w[0:D, :], (((1,), (0,)), ((), ())),
                              preferred_element_type=jnp.float32)
    acc += jax.lax.dot_general(xj, w[D:2 * D, :], (((1,), (0,)), ((), ())),
                               preferred_element_type=jnp.float32)
    acc += jax.lax.dot_general(ea, w[2 * D:3 * D, :], (((1,), (0,)), ((), ())),
                               preferred_element_type=jnp.float32)
    m = jnp.maximum(acc + b_ref[...], 0.0)
    m_ref[...] = m

    ids = rec_ref[0, 0, :]
    cbase = cstart_ref[i]
    nch = nch_ref[i]

    def chunk_body(k, carry):
        base = pl.multiple_of(cbase + k * C, C)
        rel = ids - base
        oh = (jax.lax.broadcasted_iota(jnp.int32, (C, BE), 0)
              == rel[None, :]).astype(jnp.bfloat16)
        contrib = jax.lax.dot_general(oh, m.astype(jnp.bfloat16),
                                      (((1,), (0,)), ((), ())),
                                      preferred_element_type=jnp.float32)
        aggr_ref[pl.ds(base, C), :] += contrib
        return carry

    jax.lax.fori_loop(0, nch, chunk_body, 0)


@jax.jit
def _run(x_i, x_j, recipients, edge_attr, W, b):
    rec3 = recipients.reshape(NBLK, 1, BE)
    blk_lo = recipients[::BE] // C
    blk_hi = recipients[BE - 1::BE] // C
    cstart = (blk_lo * C).astype(jnp.int32)
    nch = (blk_hi - blk_lo + 1).astype(jnp.int32)

    grid = (NBLK,)
    m, aggr = pl.pallas_call(
        _fused_kernel,
        grid=grid,
        in_specs=[
            pl.BlockSpec(memory_space=pltpu.SMEM),
            pl.BlockSpec(memory_space=pltpu.SMEM),
            pl.BlockSpec((BE, D), lambda i: (i, 0)),
            pl.BlockSpec((BE, D), lambda i: (i, 0)),
            pl.BlockSpec((BE, D), lambda i: (i, 0)),
            pl.BlockSpec((3 * D, D), lambda i: (0, 0)),
            pl.BlockSpec((1, D), lambda i: (0, 0)),
            pl.BlockSpec((1, 1, BE), lambda i: (i, 0, 0)),
        ],
        out_specs=[
            pl.BlockSpec((BE, D), lambda i: (i, 0)),
            pl.BlockSpec((NPAD, D), lambda i: (0, 0)),
        ],
        out_shape=[
            jax.ShapeDtypeStruct((E, D), jnp.float32),
            jax.ShapeDtypeStruct((NPAD, D), jnp.float32),
        ],
    )(cstart, nch, x_i, x_j, edge_attr, W, b.reshape(1, D), rec3)
    return aggr[:N], m


def kernel(x_i, x_j, recipients, edge_attr, num_segments, W, b):
    aggr, m = _run(x_i, x_j, recipients, edge_attr, W, b)
    return (aggr, m)
